# baseline (speedup 1.0000x reference)
import functools
import jax
import jax.numpy as jnp
from jax import lax
from jax.experimental import pallas as pl
from jax.experimental.pallas import tpu as pltpu
from jax.experimental.pallas import tpu_sc as plsc

N = 10000
E = 320000
NC = 2
NS = 16
NW = NC * NS
CHUNK = 80
NCH = 125
G = 25
NG = NCH // G
R = N
RPS = R // NS
RD = 10112
RDPS = RD // NS

_sc_mesh = plsc.VectorSubcoreMesh(core_axis_name="c", subcore_axis_name="s")
_sc_params = pltpu.CompilerParams(use_tc_tiling_on_sc=False)


def _deg_body(er_hbm, zeros_hbm, out_hbm, idx_v, ones_v, deg_sh, sem):
    c = lax.axis_index("c")
    s = lax.axis_index("s")
    wid = s * NC + c

    pltpu.sync_copy(zeros_hbm.at[pl.ds(s * RDPS, RDPS)],
                    deg_sh.at[pl.ds(s * RDPS, RDPS)])

    pltpu.sync_copy(er_hbm.at[1, wid], idx_v)

    @pl.loop(0, CHUNK, step=16)
    def _(i):
        ones_v[pl.ds(i, 16)] = jnp.ones((16,), jnp.float32)

    plsc.subcore_barrier()

    for k in range(G):
        pltpu.async_copy(ones_v, deg_sh.at[idx_v.at[k]], sem, add=True)

    @pl.loop(0, NCH - G)
    def _(k):
        pltpu.make_async_copy(ones_v, deg_sh.at[idx_v.at[k]], sem).wait()
        pltpu.async_copy(ones_v, deg_sh.at[idx_v.at[k + G]], sem, add=True)

    @pl.loop(NCH - G, NCH)
    def _(k):
        pltpu.make_async_copy(ones_v, deg_sh.at[idx_v.at[k]], sem).wait()

    plsc.subcore_barrier()

    pltpu.sync_copy(deg_sh.at[pl.ds(s * RDPS, RDPS)],
                    out_hbm.at[c, pl.ds(s * RDPS, RDPS)])


@functools.partial(
    pl.kernel,
    out_type=jax.ShapeDtypeStruct((NC, RD), jnp.float32),
    mesh=_sc_mesh,
    compiler_params=_sc_params,
    scratch_types=[
        pltpu.VMEM((NCH, CHUNK), jnp.int32),
        pltpu.VMEM((CHUNK,), jnp.float32),
        pltpu.VMEM_SHARED((RD,), jnp.float32),
        pltpu.SemaphoreType.DMA,
    ],
)
def _deg_kernel(er_hbm, zeros_hbm, out_hbm, idx_v, ones_v, deg_sh, sem):
    _deg_body(er_hbm, zeros_hbm, out_hbm, idx_v, ones_v, deg_sh, sem)


def _agg_body(er_hbm, y_hbm, zeros_hbm, out_hbm,
              src_v, dst_v, rows_v, acc_sh, gsem, ssem):
    c = lax.axis_index("c")
    s = lax.axis_index("s")
    wid = s * NC + c

    pltpu.sync_copy(zeros_hbm.at[pl.ds(s * RPS, RPS)],
                    acc_sh.at[pl.ds(s * RPS, RPS)])
    pltpu.sync_copy(er_hbm.at[0, wid], src_v)
    pltpu.sync_copy(er_hbm.at[1, wid], dst_v)
    plsc.subcore_barrier()

    for b in range(G):
        pltpu.async_copy(y_hbm.at[src_v.at[b]], rows_v.at[b], gsem)

    @pl.loop(0, NG)
    def _(g):
        sel = (g % 2) * G
        nsel = ((g + 1) % 2) * G

        @pl.when(g > 0)
        def _():
            for b in range(G):
                pltpu.make_async_copy(
                    rows_v.at[nsel + b],
                    acc_sh.at[dst_v.at[(g - 1) * G + b]], ssem).wait()

        @pl.when(g + 1 < NG)
        def _():
            for b in range(G):
                pltpu.async_copy(y_hbm.at[src_v.at[(g + 1) * G + b]],
                                 rows_v.at[nsel + b], gsem)

        for b in range(G):
            pltpu.make_async_copy(y_hbm.at[src_v.at[g * G + b]],
                                  rows_v.at[sel + b], gsem).wait()
            pltpu.async_copy(rows_v.at[sel + b],
                             acc_sh.at[dst_v.at[g * G + b]], ssem, add=True)

    sel_last = ((NG - 1) % 2) * G
    for b in range(G):
        pltpu.make_async_copy(rows_v.at[sel_last + b],
                              acc_sh.at[dst_v.at[(NG - 1) * G + b]],
                              ssem).wait()

    plsc.subcore_barrier()

    pltpu.sync_copy(acc_sh.at[pl.ds(s * RPS, RPS)],
                    out_hbm.at[c, pl.ds(s * RPS, RPS)])


@functools.partial(
    pl.kernel,
    out_type=jax.ShapeDtypeStruct((NC, R, 16), jnp.float32),
    mesh=_sc_mesh,
    compiler_params=_sc_params,
    scratch_types=[
        pltpu.VMEM((NCH, CHUNK), jnp.int32),
        pltpu.VMEM((NCH, CHUNK), jnp.int32),
        pltpu.VMEM((2 * G, CHUNK, 16), jnp.float32),
        pltpu.VMEM_SHARED((R, 16), jnp.float32),
        pltpu.SemaphoreType.DMA,
        pltpu.SemaphoreType.DMA,
    ],
)
def _agg_kernel(er_hbm, y_hbm, zeros_hbm, out_hbm,
                src_v, dst_v, rows_v, acc_sh, gsem, ssem):
    _agg_body(er_hbm, y_hbm, zeros_hbm, out_hbm,
              src_v, dst_v, rows_v, acc_sh, gsem, ssem)


def _t0_body(x, W1, xw_ref):
    xw_ref[...] = jnp.dot(x[...], W1[...],
                          preferred_element_type=jnp.float32)


def _t1_body(degp, xw1, dinv_ref, y1_ref):
    deg = degp[0, :N] + degp[1, :N] + 1.0
    dinv = lax.rsqrt(deg)[:, None]
    dinv_ref[...] = dinv
    y1_ref[...] = xw1[...] * dinv


def _t2_body(A1, y1, b1p, W2p, dinv_ref, y2_ref):
    dinv = dinv_ref[...]
    h1 = jax.nn.relu((A1[0] + A1[1] + y1[...]) * dinv + b1p[...])
    y2_ref[...] = jnp.dot(h1, W2p[...],
                          preferred_element_type=jnp.float32) * dinv


def _t3_body(A2, y2, b2p, W3p, b3, Wmu, bmu, Wlv, blv, eps, dinv_ref,
             mu_ref, lv_ref, z_ref, u_ref):
    dinv = dinv_ref[...]
    h2 = jax.nn.relu((A2[0] + A2[1] + y2[...]) * dinv + b2p[...])
    h = jax.nn.sigmoid(jnp.dot(h2, W3p[...],
                               preferred_element_type=jnp.float32) + b3[...])
    mu = jnp.dot(h, Wmu[...], preferred_element_type=jnp.float32) + bmu[...]
    lv = jnp.dot(h, Wlv[...], preferred_element_type=jnp.float32) + blv[...]
    z = mu + jnp.exp(0.5 * lv) * eps[...]
    mu_ref[...] = mu
    lv_ref[...] = lv
    z_ref[...] = z
    u_ref[...] = jnp.concatenate(
        [z * dinv, jnp.zeros((N, 2), jnp.float32)], axis=1)


def _t4_body(A3, u, Wd1p, bd1, dinv_ref, v_ref):
    dinv = dinv_ref[...]
    agg = (A3[0] + A3[1] + u[...]) * dinv
    d = jax.nn.relu(jnp.dot(agg, Wd1p[...],
                            preferred_element_type=jnp.float32) + bd1[...])
    v_ref[...] = d * dinv


def _t5_body(A4, v, Wd2p, bd2, dinv_ref, dec_ref):
    dinv = dinv_ref[...]
    agg = (A4[0] + A4[1] + v[...]) * dinv
    dec_ref[...] = jnp.dot(agg, Wd2p[...],
                           preferred_element_type=jnp.float32) + bd2[...]


def _dense_call(body, out_shapes, *args):
    return pl.pallas_call(
        body,
        out_shape=out_shapes,
    )(*args)


_BR = 1024
_BC = 4096


def _s_body(za_ref, zb_ref, out_ref):
    prod = lax.dot_general(za_ref[...], zb_ref[...],
                           (((1,), (1,)), ((), ())),
                           preferred_element_type=jnp.float32)
    out_ref[...] = jax.nn.sigmoid(prod)


def _s_kernel(z):
    grid = (pl.cdiv(N, _BR), pl.cdiv(N, _BC))
    return pl.pallas_call(
        _s_body,
        grid=grid,
        in_specs=[
            pl.BlockSpec((_BR, 14), lambda i, j: (i, 0)),
            pl.BlockSpec((_BC, 14), lambda i, j: (j, 0)),
        ],
        out_specs=pl.BlockSpec((_BR, _BC), lambda i, j: (i, j)),
        out_shape=jax.ShapeDtypeStruct((N, N), jnp.float32),
    )(z, z)


def kernel(x, edge_index, W1, b1, W2, b2, W3, b3, Wmu, bmu, Wlv, blv,
           Wd1, bd1, Wd2, bd2, eps):
    er = edge_index.astype(jnp.int32).reshape(2, NW, NCH, CHUNK)

    zeros_r = jnp.zeros((RD,), jnp.float32)
    zeros_acc = jnp.zeros((R, 16), jnp.float32)

    b1p = b1[None, :]
    W2p = jnp.pad(W2, ((0, 0), (0, 2)))
    b2p = jnp.pad(b2, (0, 2))[None, :]
    W3p = jnp.pad(W3, ((0, 2), (0, 0)))
    Wd1p = jnp.pad(Wd1, ((0, 2), (0, 0)))

    degp = _deg_kernel(er, zeros_r)
    xw1 = _dense_call(
        _t0_body, jax.ShapeDtypeStruct((N, 16), jnp.float32), x, W1)
    dinv, y1 = _dense_call(
        _t1_body,
        [jax.ShapeDtypeStruct((N, 1), jnp.float32),
         jax.ShapeDtypeStruct((N, 16), jnp.float32)],
        degp, xw1)

    A1 = _agg_kernel(er, y1, zeros_acc)
    y2 = _dense_call(
        _t2_body, jax.ShapeDtypeStruct((N, 16), jnp.float32),
        A1, y1, b1p, W2p, dinv)

    A2 = _agg_kernel(er, y2, zeros_acc)
    mu, lv, z, u = _dense_call(
        _t3_body,
        [jax.ShapeDtypeStruct((N, 14), jnp.float32),
         jax.ShapeDtypeStruct((N, 14), jnp.float32),
         jax.ShapeDtypeStruct((N, 14), jnp.float32),
         jax.ShapeDtypeStruct((N, 16), jnp.float32)],
        A2, y2, b2p, W3p, b3[None, :], Wmu, bmu[None, :], Wlv, blv[None, :],
        eps, dinv)

    s = _s_kernel(z)

    A3 = _agg_kernel(er, u, zeros_acc)
    v = _dense_call(
        _t4_body, jax.ShapeDtypeStruct((N, 16), jnp.float32),
        A3, u, Wd1p, bd1[None, :], dinv)

    A4 = _agg_kernel(er, v, zeros_acc)
    decoded = _dense_call(
        _t5_body, jax.ShapeDtypeStruct((N, 128), jnp.float32),
        A4, v, Wd2, bd2[None, :], dinv)

    return (s, decoded, mu, lv)


# speedup vs baseline: 1.2822x; 1.2822x over previous
import functools
import jax
import jax.numpy as jnp
from jax import lax
from jax.experimental import pallas as pl
from jax.experimental.pallas import tpu as pltpu
from jax.experimental.pallas import tpu_sc as plsc

N = 10000
E = 320000
LAT = 14
NC = 2
NS = 16
NW = NC * NS
CHUNK = 80
NCH = 125
G = 25
NG = NCH // G
RA = 10240
PR = RA * 16 // 128
RPS = RA // NS

_sc_mesh = plsc.VectorSubcoreMesh(core_axis_name="c", subcore_axis_name="s")
_sc_params = pltpu.CompilerParams(use_tc_tiling_on_sc=False)


def _deg_body(er_hbm, zeros_hbm, out_hbm, idx_v, ones_v, deg_sh, sem):
    c = lax.axis_index("c")
    s = lax.axis_index("s")
    wid = s * NC + c

    pltpu.sync_copy(zeros_hbm.at[pl.ds(s * RPS, RPS)],
                    deg_sh.at[pl.ds(s * RPS, RPS)])

    pltpu.sync_copy(er_hbm.at[1, wid], idx_v)

    @pl.loop(0, CHUNK)
    def _(i):
        ones_v[i] = jnp.ones((16,), jnp.float32)

    plsc.subcore_barrier()

    for k in range(G):
        pltpu.async_copy(ones_v, deg_sh.at[idx_v.at[k]], sem, add=True)

    @pl.loop(0, NCH - G)
    def _(k):
        pltpu.make_async_copy(ones_v, deg_sh.at[idx_v.at[k]], sem).wait()
        pltpu.async_copy(ones_v, deg_sh.at[idx_v.at[k + G]], sem, add=True)

    @pl.loop(NCH - G, NCH)
    def _(k):
        pltpu.make_async_copy(ones_v, deg_sh.at[idx_v.at[k]], sem).wait()

    plsc.subcore_barrier()

    pltpu.sync_copy(deg_sh.at[pl.ds(s * RPS, RPS)],
                    out_hbm.at[c, pl.ds(s * RPS, RPS)])


@functools.partial(
    pl.kernel,
    out_type=jax.ShapeDtypeStruct((NC, RA, 16), jnp.float32),
    mesh=_sc_mesh,
    compiler_params=_sc_params,
    scratch_types=[
        pltpu.VMEM((NCH, CHUNK), jnp.int32),
        pltpu.VMEM((CHUNK, 16), jnp.float32),
        pltpu.VMEM_SHARED((RA, 16), jnp.float32),
        pltpu.SemaphoreType.DMA,
    ],
)
def _deg_kernel(er_hbm, zeros_hbm, out_hbm, idx_v, ones_v, deg_sh, sem):
    _deg_body(er_hbm, zeros_hbm, out_hbm, idx_v, ones_v, deg_sh, sem)


def _agg_body(er_hbm, y_hbm, zeros_hbm, out_hbm,
              src_v, dst_v, rows_v, acc_sh, gsem, ssem):
    c = lax.axis_index("c")
    s = lax.axis_index("s")
    wid = s * NC + c

    pltpu.sync_copy(zeros_hbm.at[pl.ds(s * RPS, RPS)],
                    acc_sh.at[pl.ds(s * RPS, RPS)])
    pltpu.sync_copy(er_hbm.at[0, wid], src_v)
    pltpu.sync_copy(er_hbm.at[1, wid], dst_v)
    plsc.subcore_barrier()

    for b in range(G):
        pltpu.async_copy(y_hbm.at[src_v.at[b]], rows_v.at[b], gsem)

    @pl.loop(0, NG)
    def _(g):
        sel = (g % 2) * G
        nsel = ((g + 1) % 2) * G

        @pl.when(g > 0)
        def _():
            for b in range(G):
                pltpu.make_async_copy(
                    rows_v.at[nsel + b],
                    acc_sh.at[dst_v.at[(g - 1) * G + b]], ssem).wait()

        @pl.when(g + 1 < NG)
        def _():
            for b in range(G):
                pltpu.async_copy(y_hbm.at[src_v.at[(g + 1) * G + b]],
                                 rows_v.at[nsel + b], gsem)

        for b in range(G):
            pltpu.make_async_copy(y_hbm.at[src_v.at[g * G + b]],
                                  rows_v.at[sel + b], gsem).wait()
            pltpu.async_copy(rows_v.at[sel + b],
                             acc_sh.at[dst_v.at[g * G + b]], ssem, add=True)

    sel_last = ((NG - 1) % 2) * G
    for b in range(G):
        pltpu.make_async_copy(rows_v.at[sel_last + b],
                              acc_sh.at[dst_v.at[(NG - 1) * G + b]],
                              ssem).wait()

    plsc.subcore_barrier()

    pltpu.sync_copy(acc_sh.at[pl.ds(s * RPS, RPS)],
                    out_hbm.at[c, pl.ds(s * RPS, RPS)])


@functools.partial(
    pl.kernel,
    out_type=jax.ShapeDtypeStruct((NC, RA, 16), jnp.float32),
    mesh=_sc_mesh,
    compiler_params=_sc_params,
    scratch_types=[
        pltpu.VMEM((NCH, CHUNK), jnp.int32),
        pltpu.VMEM((NCH, CHUNK), jnp.int32),
        pltpu.VMEM((2 * G, CHUNK, 16), jnp.float32),
        pltpu.VMEM_SHARED((RA, 16), jnp.float32),
        pltpu.SemaphoreType.DMA,
        pltpu.SemaphoreType.DMA,
    ],
)
def _agg_kernel(er_hbm, y_hbm, zeros_hbm, out_hbm,
                src_v, dst_v, rows_v, acc_sh, gsem, ssem):
    _agg_body(er_hbm, y_hbm, zeros_hbm, out_hbm,
              src_v, dst_v, rows_v, acc_sh, gsem, ssem)


def _t0_body(xp8, W1bd, xw_ref):
    xw_ref[...] = jnp.dot(xp8[...], W1bd[...],
                          preferred_element_type=jnp.float32)


def _t1_body(degr, xw1, dinv_ref, y1_ref):
    dinv = lax.rsqrt(degr[0] + degr[1] + 1.0)
    dinv_ref[...] = dinv
    y1_ref[...] = xw1[...] * dinv


def _t2_body(A1, y1, b1t, W2bd, dinv_ref, y2_ref):
    dinv = dinv_ref[...]
    h1 = jax.nn.relu((A1[0] + A1[1] + y1[...]) * dinv + b1t[...])
    y2_ref[...] = jnp.dot(h1, W2bd[...],
                          preferred_element_type=jnp.float32) * dinv


def _t3_body(A2, y2, b2t, W3bd, b3t, Wmubd, bmut, Wlvbd, blvt, epsp, dinv_ref,
             mu_ref, lv_ref, z_ref, u_ref):
    dinv = dinv_ref[...]
    h2 = jax.nn.relu((A2[0] + A2[1] + y2[...]) * dinv + b2t[...])
    h = jax.nn.sigmoid(jnp.dot(h2, W3bd[...],
                               preferred_element_type=jnp.float32) + b3t[...])
    mu = jnp.dot(h, Wmubd[...], preferred_element_type=jnp.float32) + bmut[...]
    lv = jnp.dot(h, Wlvbd[...], preferred_element_type=jnp.float32) + blvt[...]
    z = mu + jnp.exp(0.5 * lv) * epsp[...]
    mu_ref[...] = mu
    lv_ref[...] = lv
    z_ref[...] = z
    u_ref[...] = z * dinv


def _t4_body(A3, u, Wd1bd, bd1t, dinv_ref, v_ref):
    dinv = dinv_ref[...]
    agg = (A3[0] + A3[1] + u[...]) * dinv
    d = jax.nn.relu(jnp.dot(agg, Wd1bd[...],
                            preferred_element_type=jnp.float32) + bd1t[...])
    v_ref[...] = d * dinv


def _t5_body(A4, v, Wd2bd, bd2t, dinv_ref, dec_ref):
    dinv = dinv_ref[...]
    agg = (A4[0] + A4[1] + v[...]) * dinv
    dec_ref[...] = jnp.dot(agg, Wd2bd[...],
                           preferred_element_type=jnp.float32) + bd2t[...]


def _dense_call(body, out_shapes, *args):
    return pl.pallas_call(
        body,
        out_shape=out_shapes,
    )(*args)


_BR = 2048
_BC = 2048


def _s_body(za_ref, zb_ref, out_ref):
    prod = lax.dot_general(za_ref[...], zb_ref[...],
                           (((1,), (1,)), ((), ())),
                           preferred_element_type=jnp.float32)
    out_ref[...] = jax.nn.sigmoid(prod)


def _s_kernel(z16):
    grid = (pl.cdiv(N, _BR), pl.cdiv(N, _BC))
    return pl.pallas_call(
        _s_body,
        grid=grid,
        in_specs=[
            pl.BlockSpec((_BR, 16), lambda i, j: (i, 0)),
            pl.BlockSpec((_BC, 16), lambda i, j: (j, 0)),
        ],
        out_specs=pl.BlockSpec((_BR, _BC), lambda i, j: (i, j)),
        out_shape=jax.ShapeDtypeStruct((N, N), jnp.float32),
    )(z16, z16)


def _bd8(W):
    return jnp.kron(jnp.eye(8, dtype=jnp.float32), W)


def _pad16(W):
    return jnp.pad(W, ((0, 16 - W.shape[0]), (0, 16 - W.shape[1])))


def kernel(x, edge_index, W1, b1, W2, b2, W3, b3, Wmu, bmu, Wlv, blv,
           Wd1, bd1, Wd2, bd2, eps):
    er = edge_index.astype(jnp.int32).reshape(2, NW, NCH, CHUNK)

    zeros_acc = jnp.zeros((RA, 16), jnp.float32)

    xp8 = jnp.pad(x, ((0, RA - N), (0, 0))).reshape(PR, 8 * 128)
    epsp = jnp.pad(eps, ((0, RA - N), (0, 2))).reshape(PR, 128)
    W1bd = _bd8(W1)
    W2bd = _bd8(jnp.pad(W2, ((0, 0), (0, 2))))
    W3bd = _bd8(_pad16(W3))
    Wmubd = _bd8(_pad16(Wmu))
    Wlvbd = _bd8(_pad16(Wlv))
    Wd1bd = _bd8(jnp.pad(Wd1, ((0, 2), (0, 0))))
    Wd2bd = _bd8(Wd2)
    b1t = jnp.tile(b1, 8)[None, :]
    b2t = jnp.tile(jnp.pad(b2, (0, 2)), 8)[None, :]
    b3t = jnp.tile(jnp.pad(b3, (0, 2)), 8)[None, :]
    bmut = jnp.tile(jnp.pad(bmu, (0, 2)), 8)[None, :]
    blvt = jnp.tile(jnp.pad(blv, (0, 2)), 8)[None, :]
    bd1t = jnp.tile(bd1, 8)[None, :]
    bd2t = jnp.tile(bd2, 8)[None, :]

    pk = jax.ShapeDtypeStruct((PR, 128), jnp.float32)

    degp = _deg_kernel(er, zeros_acc)
    xw1 = _dense_call(_t0_body, pk, xp8, W1bd)
    dinvp, y1 = _dense_call(_t1_body, [pk, pk], degp.reshape(NC, PR, 128), xw1)

    A1 = _agg_kernel(er, y1.reshape(RA, 16), zeros_acc)
    y2 = _dense_call(_t2_body, pk, A1.reshape(NC, PR, 128), y1, b1t, W2bd,
                     dinvp)

    A2 = _agg_kernel(er, y2.reshape(RA, 16), zeros_acc)
    mup, lvp, zp, up = _dense_call(
        _t3_body, [pk, pk, pk, pk],
        A2.reshape(NC, PR, 128), y2, b2t, W3bd, b3t, Wmubd, bmut, Wlvbd, blvt,
        epsp, dinvp)

    s = _s_kernel(zp.reshape(RA, 16))

    A3 = _agg_kernel(er, up.reshape(RA, 16), zeros_acc)
    v = _dense_call(_t4_body, pk, A3.reshape(NC, PR, 128), up, Wd1bd, bd1t,
                    dinvp)

    A4 = _agg_kernel(er, v.reshape(RA, 16), zeros_acc)
    dec = _dense_call(_t5_body, jax.ShapeDtypeStruct((PR, 8 * 128),
                                                     jnp.float32),
                      A4.reshape(NC, PR, 128), v, Wd2bd, bd2t, dinvp)

    decoded = dec.reshape(RA, 128)[:N]
    mu = mup.reshape(RA, 16)[:N, :LAT]
    logvar = lvp.reshape(RA, 16)[:N, :LAT]
    return (s, decoded, mu, logvar)


# speedup vs baseline: 1.2910x; 1.0069x over previous
import functools
import jax
import jax.numpy as jnp
from jax import lax
from jax.experimental import pallas as pl
from jax.experimental.pallas import tpu as pltpu
from jax.experimental.pallas import tpu_sc as plsc

N = 10000
E = 320000
LAT = 14
NC = 2
NS = 16
NW = NC * NS
CHUNK = 80
NCH = 125
G = 25
NG = NCH // G
RA = 10240
PR = RA * 16 // 128
RPS = RA // NS

_sc_mesh = plsc.VectorSubcoreMesh(core_axis_name="c", subcore_axis_name="s")
_sc_params = pltpu.CompilerParams(use_tc_tiling_on_sc=False)


def _deg_body(er_hbm, zeros_hbm, out_hbm, idx_v, ones_v, deg_sh, sem):
    c = lax.axis_index("c")
    s = lax.axis_index("s")
    wid = s * NC + c

    pltpu.sync_copy(zeros_hbm.at[pl.ds(s * RPS, RPS)],
                    deg_sh.at[pl.ds(s * RPS, RPS)])

    pltpu.sync_copy(er_hbm.at[1, wid], idx_v)

    @pl.loop(0, CHUNK)
    def _(i):
        ones_v[i] = jnp.ones((16,), jnp.float32)

    plsc.subcore_barrier()

    for k in range(G):
        pltpu.async_copy(ones_v, deg_sh.at[idx_v.at[k]], sem, add=True)

    @pl.loop(0, NCH - G)
    def _(k):
        pltpu.make_async_copy(ones_v, deg_sh.at[idx_v.at[k]], sem).wait()
        pltpu.async_copy(ones_v, deg_sh.at[idx_v.at[k + G]], sem, add=True)

    @pl.loop(NCH - G, NCH)
    def _(k):
        pltpu.make_async_copy(ones_v, deg_sh.at[idx_v.at[k]], sem).wait()

    plsc.subcore_barrier()

    pltpu.sync_copy(deg_sh.at[pl.ds(s * RPS, RPS)],
                    out_hbm.at[c, pl.ds(s * RPS, RPS)])


@functools.partial(
    pl.kernel,
    out_type=jax.ShapeDtypeStruct((NC, RA, 16), jnp.float32),
    mesh=_sc_mesh,
    compiler_params=_sc_params,
    scratch_types=[
        pltpu.VMEM((NCH, CHUNK), jnp.int32),
        pltpu.VMEM((CHUNK, 16), jnp.float32),
        pltpu.VMEM_SHARED((RA, 16), jnp.float32),
        pltpu.SemaphoreType.DMA,
    ],
)
def _deg_kernel(er_hbm, zeros_hbm, out_hbm, idx_v, ones_v, deg_sh, sem):
    _deg_body(er_hbm, zeros_hbm, out_hbm, idx_v, ones_v, deg_sh, sem)


def _agg_body(er_hbm, y_hbm, zeros_hbm, out_hbm,
              src_v, dst_v, rows_v, acc_sh, gsem, ssem):
    c = lax.axis_index("c")
    s = lax.axis_index("s")
    wid = s * NC + c

    pltpu.sync_copy(zeros_hbm.at[pl.ds(s * RPS, RPS)],
                    acc_sh.at[pl.ds(s * RPS, RPS)])
    pltpu.sync_copy(er_hbm.at[0, wid], src_v)
    pltpu.sync_copy(er_hbm.at[1, wid], dst_v)
    plsc.subcore_barrier()

    for b in range(G):
        pltpu.async_copy(y_hbm.at[src_v.at[b]], rows_v.at[b], gsem)

    @pl.loop(0, NG)
    def _(g):
        sel = (g % 2) * G
        nsel = ((g + 1) % 2) * G

        @pl.when(g > 0)
        def _():
            for b in range(G):
                pltpu.make_async_copy(
                    rows_v.at[nsel + b],
                    acc_sh.at[dst_v.at[(g - 1) * G + b]], ssem).wait()

        @pl.when(g + 1 < NG)
        def _():
            for b in range(G):
                pltpu.async_copy(y_hbm.at[src_v.at[(g + 1) * G + b]],
                                 rows_v.at[nsel + b], gsem)

        for b in range(G):
            pltpu.make_async_copy(y_hbm.at[src_v.at[g * G + b]],
                                  rows_v.at[sel + b], gsem).wait()
            pltpu.async_copy(rows_v.at[sel + b],
                             acc_sh.at[dst_v.at[g * G + b]], ssem, add=True)

    sel_last = ((NG - 1) % 2) * G
    for b in range(G):
        pltpu.make_async_copy(rows_v.at[sel_last + b],
                              acc_sh.at[dst_v.at[(NG - 1) * G + b]],
                              ssem).wait()

    plsc.subcore_barrier()

    pltpu.sync_copy(acc_sh.at[pl.ds(s * RPS, RPS)],
                    out_hbm.at[c, pl.ds(s * RPS, RPS)])


@functools.partial(
    pl.kernel,
    out_type=jax.ShapeDtypeStruct((NC, RA, 16), jnp.float32),
    mesh=_sc_mesh,
    compiler_params=_sc_params,
    scratch_types=[
        pltpu.VMEM((NCH, CHUNK), jnp.int32),
        pltpu.VMEM((NCH, CHUNK), jnp.int32),
        pltpu.VMEM((2 * G, CHUNK, 16), jnp.float32),
        pltpu.VMEM_SHARED((RA, 16), jnp.float32),
        pltpu.SemaphoreType.DMA,
        pltpu.SemaphoreType.DMA,
    ],
)
def _agg_kernel(er_hbm, y_hbm, zeros_hbm, out_hbm,
                src_v, dst_v, rows_v, acc_sh, gsem, ssem):
    _agg_body(er_hbm, y_hbm, zeros_hbm, out_hbm,
              src_v, dst_v, rows_v, acc_sh, gsem, ssem)


def _t0_body(xp8, W1bd, xw_ref):
    xw_ref[...] = jnp.dot(xp8[...], W1bd[...],
                          preferred_element_type=jnp.float32)


def _t1_body(degr, xw1, dinv_ref, y1_ref):
    dinv = lax.rsqrt(degr[0] + degr[1] + 1.0)
    dinv_ref[...] = dinv
    y1_ref[...] = xw1[...] * dinv


def _t2_body(A1, y1, b1t, W2bd, dinv_ref, y2_ref):
    dinv = dinv_ref[...]
    h1 = jax.nn.relu((A1[0] + A1[1] + y1[...]) * dinv + b1t[...])
    y2_ref[...] = jnp.dot(h1, W2bd[...],
                          preferred_element_type=jnp.float32) * dinv


def _t3_body(A2, y2, b2t, W3bd, b3t, Wmubd, bmut, Wlvbd, blvt, epsp, dinv_ref,
             mu_ref, lv_ref, z_ref, u_ref):
    dinv = dinv_ref[...]
    h2 = jax.nn.relu((A2[0] + A2[1] + y2[...]) * dinv + b2t[...])
    h = jax.nn.sigmoid(jnp.dot(h2, W3bd[...],
                               preferred_element_type=jnp.float32) + b3t[...])
    mu = jnp.dot(h, Wmubd[...], preferred_element_type=jnp.float32) + bmut[...]
    lv = jnp.dot(h, Wlvbd[...], preferred_element_type=jnp.float32) + blvt[...]
    z = mu + jnp.exp(0.5 * lv) * epsp[...]
    mu_ref[...] = mu
    lv_ref[...] = lv
    z_ref[...] = z
    u_ref[...] = z * dinv


def _t4_body(A3, u, Wd1bd, bd1t, dinv_ref, v_ref):
    dinv = dinv_ref[...]
    agg = (A3[0] + A3[1] + u[...]) * dinv
    d = jax.nn.relu(jnp.dot(agg, Wd1bd[...],
                            preferred_element_type=jnp.float32) + bd1t[...])
    v_ref[...] = d * dinv


def _t5_body(A4, v, Wd2bd, bd2t, dinv_ref, dec_ref):
    dinv = dinv_ref[...]
    agg = (A4[0] + A4[1] + v[...]) * dinv
    dec_ref[...] = jnp.dot(agg, Wd2bd[...],
                           preferred_element_type=jnp.float32) + bd2t[...]


def _dense_call(body, out_shapes, *args):
    return pl.pallas_call(
        body,
        out_shape=out_shapes,
    )(*args)


_BR = 2560
_BC = 2048


def _s_body(za_ref, zb_ref, out_ref):
    prod = lax.dot_general(za_ref[...], zb_ref[...],
                           (((1,), (1,)), ((), ())),
                           preferred_element_type=jnp.float32)
    out_ref[...] = jax.nn.sigmoid(prod)


def _s_kernel(z16):
    grid = (pl.cdiv(N, _BR), pl.cdiv(N, _BC))
    return pl.pallas_call(
        _s_body,
        grid=grid,
        in_specs=[
            pl.BlockSpec((_BR, 16), lambda i, j: (i, 0)),
            pl.BlockSpec((_BC, 16), lambda i, j: (j, 0)),
        ],
        out_specs=pl.BlockSpec((_BR, _BC), lambda i, j: (i, j)),
        out_shape=jax.ShapeDtypeStruct((N, N), jnp.float32),
    )(z16, z16)


def _bd8(W):
    return jnp.kron(jnp.eye(8, dtype=jnp.float32), W)


def _pad16(W):
    return jnp.pad(W, ((0, 16 - W.shape[0]), (0, 16 - W.shape[1])))


def kernel(x, edge_index, W1, b1, W2, b2, W3, b3, Wmu, bmu, Wlv, blv,
           Wd1, bd1, Wd2, bd2, eps):
    er = edge_index.astype(jnp.int32).reshape(2, NW, NCH, CHUNK)

    zeros_acc = jnp.zeros((RA, 16), jnp.float32)

    xp8 = jnp.pad(x, ((0, RA - N), (0, 0))).reshape(PR, 8 * 128)
    epsp = jnp.pad(eps, ((0, RA - N), (0, 2))).reshape(PR, 128)
    W1bd = _bd8(W1)
    W2bd = _bd8(jnp.pad(W2, ((0, 0), (0, 2))))
    W3bd = _bd8(_pad16(W3))
    Wmubd = _bd8(_pad16(Wmu))
    Wlvbd = _bd8(_pad16(Wlv))
    Wd1bd = _bd8(jnp.pad(Wd1, ((0, 2), (0, 0))))
    Wd2bd = _bd8(Wd2)
    b1t = jnp.tile(b1, 8)[None, :]
    b2t = jnp.tile(jnp.pad(b2, (0, 2)), 8)[None, :]
    b3t = jnp.tile(jnp.pad(b3, (0, 2)), 8)[None, :]
    bmut = jnp.tile(jnp.pad(bmu, (0, 2)), 8)[None, :]
    blvt = jnp.tile(jnp.pad(blv, (0, 2)), 8)[None, :]
    bd1t = jnp.tile(bd1, 8)[None, :]
    bd2t = jnp.tile(bd2, 8)[None, :]

    pk = jax.ShapeDtypeStruct((PR, 128), jnp.float32)

    degp = _deg_kernel(er, zeros_acc)
    xw1 = _dense_call(_t0_body, pk, xp8, W1bd)
    dinvp, y1 = _dense_call(_t1_body, [pk, pk], degp.reshape(NC, PR, 128), xw1)

    A1 = _agg_kernel(er, y1.reshape(RA, 16), zeros_acc)
    y2 = _dense_call(_t2_body, pk, A1.reshape(NC, PR, 128), y1, b1t, W2bd,
                     dinvp)

    A2 = _agg_kernel(er, y2.reshape(RA, 16), zeros_acc)
    mup, lvp, zp, up = _dense_call(
        _t3_body, [pk, pk, pk, pk],
        A2.reshape(NC, PR, 128), y2, b2t, W3bd, b3t, Wmubd, bmut, Wlvbd, blvt,
        epsp, dinvp)

    s = _s_kernel(zp.reshape(RA, 16))

    A3 = _agg_kernel(er, up.reshape(RA, 16), zeros_acc)
    v = _dense_call(_t4_body, pk, A3.reshape(NC, PR, 128), up, Wd1bd, bd1t,
                    dinvp)

    A4 = _agg_kernel(er, v.reshape(RA, 16), zeros_acc)
    dec = _dense_call(_t5_body, jax.ShapeDtypeStruct((PR, 8 * 128),
                                                     jnp.float32),
                      A4.reshape(NC, PR, 128), v, Wd2bd, bd2t, dinvp)

    decoded = dec.reshape(RA, 128)[:N]
    mu = mup.reshape(RA, 16)[:N, :LAT]
    logvar = lvp.reshape(RA, 16)[:N, :LAT]
    return (s, decoded, mu, logvar)
